# baseline (speedup 1.0000x reference)
import functools

import jax
import jax.numpy as jnp
from jax import lax
from jax.experimental import pallas as pl
from jax.experimental.pallas import tpu as pltpu
from jax.experimental.pallas import tpu_sc as plsc

NC = 2
NS = 16
NW = NC * NS
L = 16

CHUNK = 128
NSLOT = 4


def _sc_body(M, B, R, mem_hbm, idx_hbm, val_hbm, out_hbm,
             idx_v, row_buf, pos_buf, table, rot16, keep16,
             cb0, cb1, cb2, cb3,
             sg0, sg1, sg2, sg3, ss0, ss1, ss2, ss3):
    tail = M - (NW - 1) * R
    n_full_last = tail // CHUNK
    n_full_main = R // CHUNK
    t_main = R - n_full_main * CHUNK
    t_last = tail - n_full_last * CHUNK

    wid = lax.axis_index("s") * NC + lax.axis_index("c")
    lo = wid * R
    hi = jnp.minimum(lo + R, M)
    iota = lax.iota(jnp.int32, L)

    cbufs = (cb0, cb1, cb2, cb3)
    gsems = (sg0, sg1, sg2, sg3)
    ssems = (ss0, ss1, ss2, ss3)

    nfull = jnp.where(wid == NW - 1, n_full_last, n_full_main)

    def cg(c, s):
        return pltpu.make_async_copy(
            mem_hbm.at[pl.ds(lo + c * CHUNK, CHUNK)], cbufs[s], gsems[s])

    def cs(c, s):
        return pltpu.make_async_copy(
            cbufs[s], out_hbm.at[pl.ds(lo + c * CHUNK, CHUNK)], ssems[s])

    rounds = (max(n_full_main, n_full_last) + 2 * NSLOT - 1) // NSLOT + 1

    def copy_round(r, _):
        for j in range(NSLOT):
            c = r * NSLOT + j
            s_cur = j
            s_prev = (j - 1) % NSLOT

            @pl.when((c >= NSLOT) & (c - NSLOT < nfull))
            def _():
                cs(c - NSLOT, s_cur).wait()

            @pl.when(c < nfull)
            def _():
                cg(c, s_cur).start()

            @pl.when((c >= 1) & (c - 1 < nfull))
            def _():
                cg(c - 1, s_prev).wait()
                cs(c - 1, s_prev).start()

        return 0

    lax.fori_loop(0, rounds, copy_round, 0)

    def tail_copy(n_full, t_rows):
        base = lo + n_full * CHUNK
        pltpu.sync_copy(mem_hbm.at[pl.ds(base, t_rows)],
                        cb0.at[pl.ds(0, t_rows)])
        pltpu.sync_copy(cb0.at[pl.ds(0, t_rows)],
                        out_hbm.at[pl.ds(base, t_rows)])

    if t_main > 0:
        @pl.when(wid < NW - 1)
        def _():
            tail_copy(n_full_main, t_main)

    if t_last > 0:
        @pl.when(wid == NW - 1)
        def _():
            tail_copy(n_full_last, t_last)

    pltpu.sync_copy(idx_hbm, idx_v)

    def one_chunk(base, cursor):
        x = plsc.load_gather(idx_v, [base + iota])
        pos = base + iota
        m = (x >= lo) & (x < hi)
        mi = m.astype(jnp.int32)
        cnt = jnp.sum(mi)

        @pl.when(cnt > 0)
        def _():
            dest = cursor + jnp.cumsum(mi) - 1
            plsc.store_scatter(
                row_buf, [dest >> 7, dest & (CHUNK - 1)], x, mask=m)
            plsc.store_scatter(
                pos_buf, [dest >> 7, dest & (CHUNK - 1)], pos, mask=m)
            keep16[...] = mi

            @pl.when(cnt > 1)
            def _():
                rot16[...] = jnp.where(m, x, -1)
                dup = jnp.zeros((L,), jnp.bool_)
                for r in range(1, L):
                    y = plsc.load_gather(rot16, [(iota + r) & (L - 1)])
                    later = (iota + r) < L
                    dup = dup | (m & later & (y == x))
                keep16[...] = jnp.where(dup, 0, mi)

            keep = keep16[...] > 0
            plsc.store_scatter(table, [x - lo], pos, mask=keep)

        return cursor + cnt

    def scan_body(i, cursor):
        cursor = one_chunk(i * 2 * L, cursor)
        cursor = one_chunk(i * 2 * L + L, cursor)
        return cursor

    n_cand = lax.fori_loop(0, B // (2 * L), scan_body, jnp.int32(0))

    def win_body(c, carry):
        wcur, pad_row, pad_pos = carry
        fl = c * L + iota
        valid = fl < n_cand
        fb, fc = fl >> 7, fl & (CHUNK - 1)
        x = plsc.load_gather(row_buf, [fb, fc], mask=valid)
        p = plsc.load_gather(pos_buf, [fb, fc], mask=valid)
        w = plsc.load_gather(table, [jnp.where(valid, x - lo, 0)], mask=valid)
        keep = valid & (w == p)
        ki = keep.astype(jnp.int32)
        kcnt = jnp.sum(ki)
        dest = wcur + jnp.cumsum(ki) - 1
        plsc.store_scatter(row_buf, [dest >> 7, dest & (CHUNK - 1)], x, mask=keep)
        plsc.store_scatter(pos_buf, [dest >> 7, dest & (CHUNK - 1)], p, mask=keep)
        lmax = jnp.max(jnp.where(keep, iota, -1))
        sel = keep & (iota == lmax)
        pr = jnp.max(jnp.where(sel, x, -1))
        pp = jnp.max(jnp.where(sel, p, -1))
        pad_row = jnp.where(kcnt > 0, pr, pad_row)
        pad_pos = jnp.where(kcnt > 0, pp, pad_pos)
        return wcur + kcnt, pad_row, pad_pos

    n_win, pad_row, pad_pos = lax.fori_loop(
        0, pl.cdiv(n_cand, L), win_body,
        (jnp.int32(0), jnp.int32(0), jnp.int32(0)))

    n_tot = pl.cdiv(n_win, CHUNK) * CHUNK

    def pad_body(c, _):
        e = n_win + c * L + iota
        mm = e < n_tot
        plsc.store_scatter(
            row_buf, [e >> 7, e & (CHUNK - 1)],
            jnp.full((L,), pad_row, jnp.int32), mask=mm)
        plsc.store_scatter(
            pos_buf, [e >> 7, e & (CHUNK - 1)],
            jnp.full((L,), pad_pos, jnp.int32), mask=mm)
        return 0

    lax.fori_loop(0, pl.cdiv(n_tot - n_win, L), pad_body, 0)

    nb = n_tot // CHUNK
    rows0, rows1 = cb0, cb1
    sem_g0, sem_g1, sem_s0, sem_s1 = sg0, sg1, ss0, ss1

    def gather(b, rows, sem):
        pltpu.make_async_copy(
            val_hbm.at[pos_buf.at[b]], rows, sem).start()

    @pl.when(nb > 0)
    def _():
        gather(0, rows0, sem_g0)

    def pair_body(i, _):
        b = i * 2
        pltpu.make_async_copy(
            val_hbm.at[pos_buf.at[b]], rows0, sem_g0).wait()
        s0 = pltpu.make_async_copy(
            rows0, out_hbm.at[row_buf.at[b]], sem_s0)
        s0.start()

        @pl.when(b + 1 < nb)
        def _():
            gather(b + 1, rows1, sem_g1)

        s0.wait()

        @pl.when(b + 1 < nb)
        def _():
            pltpu.make_async_copy(
                val_hbm.at[pos_buf.at[b + 1]], rows1, sem_g1).wait()
            s1 = pltpu.make_async_copy(
                rows1, out_hbm.at[row_buf.at[b + 1]], sem_s1)
            s1.start()

            @pl.when(b + 2 < nb)
            def _():
                gather(b + 2, rows0, sem_g0)

            s1.wait()

        return 0

    lax.fori_loop(0, pl.cdiv(nb, 2), pair_body, 0)


def kernel(mem, idx, val):
    M, D = mem.shape
    B, _ = val.shape
    assert B % (2 * L) == 0
    R = ((M + NW - 1) // NW + 7) // 8 * 8
    assert 0 < M - (NW - 1) * R <= R and M % 8 == 0
    NB_MAX = B // CHUNK

    mesh = plsc.VectorSubcoreMesh(
        core_axis_name="c", subcore_axis_name="s", num_cores=NC)

    sc = pl.kernel(
        functools.partial(_sc_body, M, B, R),
        out_type=jax.ShapeDtypeStruct((M, D), jnp.float32),
        mesh=mesh,
        compiler_params=pltpu.CompilerParams(needs_layout_passes=False),
        scratch_types=[
            pltpu.VMEM((B,), jnp.int32),
            pltpu.VMEM((NB_MAX, CHUNK), jnp.int32),
            pltpu.VMEM((NB_MAX, CHUNK), jnp.int32),
            pltpu.VMEM((R,), jnp.int32),
            pltpu.VMEM((L,), jnp.int32),
            pltpu.VMEM((L,), jnp.int32),
            pltpu.VMEM((CHUNK, D), jnp.float32),
            pltpu.VMEM((CHUNK, D), jnp.float32),
            pltpu.VMEM((CHUNK, D), jnp.float32),
            pltpu.VMEM((CHUNK, D), jnp.float32),
        ] + [pltpu.SemaphoreType.DMA] * 8,
    )
    return sc(mem, idx, val)


# speedup vs baseline: 1.0808x; 1.0808x over previous
import functools

import jax
import jax.numpy as jnp
from jax import lax
from jax.experimental import pallas as pl
from jax.experimental.pallas import tpu as pltpu
from jax.experimental.pallas import tpu_sc as plsc

NC = 2
NS = 16
NW = NC * NS
L = 16

CHUNK = 128
NSLOT = 4


def _sc_body(M, B, R, mem_hbm, idx_hbm, val_hbm, out_hbm,
             idx_v, row_buf, pos_buf, table, rot16, keep16,
             cb0, cb1, cb2, cb3,
             sg0, sg1, sg2, sg3, ss0, ss1, ss2, ss3):
    tail = M - (NW - 1) * R
    n_full_last = tail // CHUNK
    n_full_main = R // CHUNK
    t_main = R - n_full_main * CHUNK
    t_last = tail - n_full_last * CHUNK

    wid = lax.axis_index("s") * NC + lax.axis_index("c")
    lo = wid * R
    hi = jnp.minimum(lo + R, M)
    iota = lax.iota(jnp.int32, L)

    cbufs = (cb0, cb1, cb2, cb3)
    gsems = (sg0, sg1, sg2, sg3)
    ssems = (ss0, ss1, ss2, ss3)

    nfull = jnp.where(wid == NW - 1, n_full_last, n_full_main)

    def cg(c, s):
        return pltpu.make_async_copy(
            mem_hbm.at[pl.ds(lo + c * CHUNK, CHUNK)], cbufs[s], gsems[s])

    def cs(c, s):
        return pltpu.make_async_copy(
            cbufs[s], out_hbm.at[pl.ds(lo + c * CHUNK, CHUNK)], ssems[s])

    rounds = (max(n_full_main, n_full_last) + 2 * NSLOT - 1) // NSLOT + 1

    def copy_round(r):
        for j in range(NSLOT):
            c = r * NSLOT + j
            s_cur = j
            s_prev = (j - 1) % NSLOT

            @pl.when((c >= NSLOT) & (c - NSLOT < nfull))
            def _():
                cs(c - NSLOT, s_cur).wait()

            @pl.when(c < nfull)
            def _():
                cg(c, s_cur).start()

            @pl.when((c >= 1) & (c - 1 < nfull))
            def _():
                cg(c - 1, s_prev).wait()
                cs(c - 1, s_prev).start()

    pltpu.sync_copy(idx_hbm, idx_v)

    def one_chunk(base, cursor):
        x = plsc.load_gather(idx_v, [base + iota])
        pos = base + iota
        m = (x >= lo) & (x < hi)
        mi = m.astype(jnp.int32)
        cnt = jnp.sum(mi)

        @pl.when(cnt > 0)
        def _():
            dest = cursor + jnp.cumsum(mi) - 1
            plsc.store_scatter(
                row_buf, [dest >> 7, dest & (CHUNK - 1)], x, mask=m)
            plsc.store_scatter(
                pos_buf, [dest >> 7, dest & (CHUNK - 1)], pos, mask=m)
            keep16[...] = mi

            @pl.when(cnt > 1)
            def _():
                rot16[...] = jnp.where(m, x, -1)
                dup = jnp.zeros((L,), jnp.bool_)
                for r in range(1, L):
                    y = plsc.load_gather(rot16, [(iota + r) & (L - 1)])
                    later = (iota + r) < L
                    dup = dup | (m & later & (y == x))
                keep16[...] = jnp.where(dup, 0, mi)

            keep = keep16[...] > 0
            plsc.store_scatter(table, [x - lo], pos, mask=keep)

        return cursor + cnt

    NIT = B // (2 * L)
    K_INT = max(1, NIT // (rounds + 1))
    assert (NIT - 1) // K_INT >= rounds - 1

    def scan_body(i, cursor):
        @pl.when(i % K_INT == 0)
        def _():
            copy_round(i // K_INT)

        cursor = one_chunk(i * 2 * L, cursor)
        cursor = one_chunk(i * 2 * L + L, cursor)
        return cursor

    n_cand = lax.fori_loop(0, NIT, scan_body, jnp.int32(0))

    def tail_copy(n_full, t_rows):
        base = lo + n_full * CHUNK
        pltpu.sync_copy(mem_hbm.at[pl.ds(base, t_rows)],
                        cb0.at[pl.ds(0, t_rows)])
        pltpu.sync_copy(cb0.at[pl.ds(0, t_rows)],
                        out_hbm.at[pl.ds(base, t_rows)])

    if t_main > 0:
        @pl.when(wid < NW - 1)
        def _():
            tail_copy(n_full_main, t_main)

    if t_last > 0:
        @pl.when(wid == NW - 1)
        def _():
            tail_copy(n_full_last, t_last)

    def win_body(c, carry):
        wcur, pad_row, pad_pos = carry
        fl = c * L + iota
        valid = fl < n_cand
        fb, fc = fl >> 7, fl & (CHUNK - 1)
        x = plsc.load_gather(row_buf, [fb, fc], mask=valid)
        p = plsc.load_gather(pos_buf, [fb, fc], mask=valid)
        w = plsc.load_gather(table, [jnp.where(valid, x - lo, 0)], mask=valid)
        keep = valid & (w == p)
        ki = keep.astype(jnp.int32)
        kcnt = jnp.sum(ki)
        dest = wcur + jnp.cumsum(ki) - 1
        plsc.store_scatter(row_buf, [dest >> 7, dest & (CHUNK - 1)], x, mask=keep)
        plsc.store_scatter(pos_buf, [dest >> 7, dest & (CHUNK - 1)], p, mask=keep)
        lmax = jnp.max(jnp.where(keep, iota, -1))
        sel = keep & (iota == lmax)
        pr = jnp.max(jnp.where(sel, x, -1))
        pp = jnp.max(jnp.where(sel, p, -1))
        pad_row = jnp.where(kcnt > 0, pr, pad_row)
        pad_pos = jnp.where(kcnt > 0, pp, pad_pos)
        return wcur + kcnt, pad_row, pad_pos

    n_win, pad_row, pad_pos = lax.fori_loop(
        0, pl.cdiv(n_cand, L), win_body,
        (jnp.int32(0), jnp.int32(0), jnp.int32(0)))

    n_tot = pl.cdiv(n_win, CHUNK) * CHUNK

    def pad_body(c, _):
        e = n_win + c * L + iota
        mm = e < n_tot
        plsc.store_scatter(
            row_buf, [e >> 7, e & (CHUNK - 1)],
            jnp.full((L,), pad_row, jnp.int32), mask=mm)
        plsc.store_scatter(
            pos_buf, [e >> 7, e & (CHUNK - 1)],
            jnp.full((L,), pad_pos, jnp.int32), mask=mm)
        return 0

    lax.fori_loop(0, pl.cdiv(n_tot - n_win, L), pad_body, 0)

    nb = n_tot // CHUNK
    rows0, rows1 = cb0, cb1
    sem_g0, sem_g1, sem_s0, sem_s1 = sg0, sg1, ss0, ss1

    def gather(b, rows, sem):
        pltpu.make_async_copy(
            val_hbm.at[pos_buf.at[b]], rows, sem).start()

    @pl.when(nb > 0)
    def _():
        gather(0, rows0, sem_g0)

    def pair_body(i, _):
        b = i * 2
        pltpu.make_async_copy(
            val_hbm.at[pos_buf.at[b]], rows0, sem_g0).wait()
        s0 = pltpu.make_async_copy(
            rows0, out_hbm.at[row_buf.at[b]], sem_s0)
        s0.start()

        @pl.when(b + 1 < nb)
        def _():
            gather(b + 1, rows1, sem_g1)

        s0.wait()

        @pl.when(b + 1 < nb)
        def _():
            pltpu.make_async_copy(
                val_hbm.at[pos_buf.at[b + 1]], rows1, sem_g1).wait()
            s1 = pltpu.make_async_copy(
                rows1, out_hbm.at[row_buf.at[b + 1]], sem_s1)
            s1.start()

            @pl.when(b + 2 < nb)
            def _():
                gather(b + 2, rows0, sem_g0)

            s1.wait()

        return 0

    lax.fori_loop(0, pl.cdiv(nb, 2), pair_body, 0)


def kernel(mem, idx, val):
    M, D = mem.shape
    B, _ = val.shape
    assert B % (2 * L) == 0
    R = ((M + NW - 1) // NW + 7) // 8 * 8
    assert 0 < M - (NW - 1) * R <= R and M % 8 == 0
    NB_MAX = B // CHUNK

    mesh = plsc.VectorSubcoreMesh(
        core_axis_name="c", subcore_axis_name="s", num_cores=NC)

    sc = pl.kernel(
        functools.partial(_sc_body, M, B, R),
        out_type=jax.ShapeDtypeStruct((M, D), jnp.float32),
        mesh=mesh,
        compiler_params=pltpu.CompilerParams(needs_layout_passes=False),
        scratch_types=[
            pltpu.VMEM((B,), jnp.int32),
            pltpu.VMEM((NB_MAX, CHUNK), jnp.int32),
            pltpu.VMEM((NB_MAX, CHUNK), jnp.int32),
            pltpu.VMEM((R,), jnp.int32),
            pltpu.VMEM((L,), jnp.int32),
            pltpu.VMEM((L,), jnp.int32),
            pltpu.VMEM((CHUNK, D), jnp.float32),
            pltpu.VMEM((CHUNK, D), jnp.float32),
            pltpu.VMEM((CHUNK, D), jnp.float32),
            pltpu.VMEM((CHUNK, D), jnp.float32),
        ] + [pltpu.SemaphoreType.DMA] * 8,
    )
    return sc(mem, idx, val)


# speedup vs baseline: 1.1613x; 1.0745x over previous
import functools

import jax
import jax.numpy as jnp
from jax import lax
from jax.experimental import pallas as pl
from jax.experimental.pallas import tpu as pltpu
from jax.experimental.pallas import tpu_sc as plsc

NC = 2
NS = 16
NW = NC * NS
L = 16

CHUNK = 128
NSLOT = 2
CCH = 224


def _sc_body(M, B, R, mem_hbm, idx_hbm, val_hbm, out_hbm,
             idx_v, row_buf, pos_buf, table, rot16, keep16,
             cb0, cb1,
             sg0, sg1, ss0, ss1):
    tail = M - (NW - 1) * R
    n_full_last = tail // CCH
    n_full_main = R // CCH
    t_main = R - n_full_main * CCH
    t_last = tail - n_full_last * CCH

    wid = lax.axis_index("s") * NC + lax.axis_index("c")
    lo = wid * R
    hi = jnp.minimum(lo + R, M)
    iota = lax.iota(jnp.int32, L)

    cbufs = (cb0, cb1)
    gsems = (sg0, sg1)
    ssems = (ss0, ss1)

    if n_full_main == n_full_last:
        nfull = n_full_main
    else:
        nfull = jnp.where(wid == NW - 1, n_full_last, n_full_main)

    def cg(c, s):
        return pltpu.make_async_copy(
            mem_hbm.at[pl.ds(lo + c * CCH, CCH)], cbufs[s], gsems[s])

    def cs(c, s):
        return pltpu.make_async_copy(
            cbufs[s], out_hbm.at[pl.ds(lo + c * CCH, CCH)], ssems[s])

    rounds = (max(n_full_main, n_full_last) + 2 * NSLOT - 1) // NSLOT + 1

    def copy_round(r):
        for j in range(NSLOT):
            c = r * NSLOT + j
            s_cur = j
            s_prev = (j - 1) % NSLOT

            @pl.when((c >= NSLOT) & (c - NSLOT < nfull))
            def _():
                cs(c - NSLOT, s_cur).wait()

            @pl.when(c < nfull)
            def _():
                cg(c, s_cur).start()

            @pl.when((c >= 1) & (c - 1 < nfull))
            def _():
                cg(c - 1, s_prev).wait()
                cs(c - 1, s_prev).start()

    pltpu.sync_copy(idx_hbm, idx_v)

    def one_chunk(base, cursor):
        x = plsc.load_gather(idx_v, [base + iota])
        pos = base + iota
        m = (x >= lo) & (x < hi)
        mi = m.astype(jnp.int32)
        cnt = jnp.sum(mi)

        @pl.when(cnt > 0)
        def _():
            dest = cursor + jnp.cumsum(mi) - 1
            plsc.store_scatter(
                row_buf, [dest >> 7, dest & (CHUNK - 1)], x, mask=m)
            plsc.store_scatter(
                pos_buf, [dest >> 7, dest & (CHUNK - 1)], pos, mask=m)
            keep16[...] = mi

            @pl.when(cnt > 1)
            def _():
                rot16[...] = jnp.where(m, x, -1)
                dup = jnp.zeros((L,), jnp.bool_)
                for r in range(1, L):
                    y = plsc.load_gather(rot16, [(iota + r) & (L - 1)])
                    later = (iota + r) < L
                    dup = dup | (m & later & (y == x))
                keep16[...] = jnp.where(dup, 0, mi)

            keep = keep16[...] > 0
            plsc.store_scatter(table, [x - lo], pos, mask=keep)

        return cursor + cnt

    NIT = B // (2 * L)
    K_INT = max(1, NIT // (rounds + 1))
    assert (NIT - 1) // K_INT >= rounds - 1

    def scan_body(i, cursor):
        @pl.when(i % K_INT == 0)
        def _():
            copy_round(i // K_INT)

        cursor = one_chunk(i * 2 * L, cursor)
        cursor = one_chunk(i * 2 * L + L, cursor)
        return cursor

    n_cand = lax.fori_loop(0, NIT, scan_body, jnp.int32(0))

    def tail_copy(n_full, t_rows):
        base = lo + n_full * CCH
        pltpu.sync_copy(mem_hbm.at[pl.ds(base, t_rows)],
                        cb0.at[pl.ds(0, t_rows)])
        pltpu.sync_copy(cb0.at[pl.ds(0, t_rows)],
                        out_hbm.at[pl.ds(base, t_rows)])

    if t_main > 0:
        @pl.when(wid < NW - 1)
        def _():
            tail_copy(n_full_main, t_main)

    if t_last > 0:
        @pl.when(wid == NW - 1)
        def _():
            tail_copy(n_full_last, t_last)

    def win_body(c, carry):
        wcur, pad_row, pad_pos = carry
        fl = c * L + iota
        valid = fl < n_cand
        fb, fc = fl >> 7, fl & (CHUNK - 1)
        x = plsc.load_gather(row_buf, [fb, fc], mask=valid)
        p = plsc.load_gather(pos_buf, [fb, fc], mask=valid)
        w = plsc.load_gather(table, [jnp.where(valid, x - lo, 0)], mask=valid)
        keep = valid & (w == p)
        ki = keep.astype(jnp.int32)
        kcnt = jnp.sum(ki)
        dest = wcur + jnp.cumsum(ki) - 1
        plsc.store_scatter(row_buf, [dest >> 7, dest & (CHUNK - 1)], x, mask=keep)
        plsc.store_scatter(pos_buf, [dest >> 7, dest & (CHUNK - 1)], p, mask=keep)
        lmax = jnp.max(jnp.where(keep, iota, -1))
        sel = keep & (iota == lmax)
        pr = jnp.max(jnp.where(sel, x, -1))
        pp = jnp.max(jnp.where(sel, p, -1))
        pad_row = jnp.where(kcnt > 0, pr, pad_row)
        pad_pos = jnp.where(kcnt > 0, pp, pad_pos)
        return wcur + kcnt, pad_row, pad_pos

    n_win, pad_row, pad_pos = lax.fori_loop(
        0, pl.cdiv(n_cand, L), win_body,
        (jnp.int32(0), jnp.int32(0), jnp.int32(0)))

    n_tot = pl.cdiv(n_win, CHUNK) * CHUNK

    def pad_body(c, _):
        e = n_win + c * L + iota
        mm = e < n_tot
        plsc.store_scatter(
            row_buf, [e >> 7, e & (CHUNK - 1)],
            jnp.full((L,), pad_row, jnp.int32), mask=mm)
        plsc.store_scatter(
            pos_buf, [e >> 7, e & (CHUNK - 1)],
            jnp.full((L,), pad_pos, jnp.int32), mask=mm)
        return 0

    lax.fori_loop(0, pl.cdiv(n_tot - n_win, L), pad_body, 0)

    nb = n_tot // CHUNK
    rows0 = cb0.at[pl.ds(0, CHUNK)]
    rows1 = cb1.at[pl.ds(0, CHUNK)]
    sem_g0, sem_g1, sem_s0, sem_s1 = sg0, sg1, ss0, ss1

    def gather(b, rows, sem):
        pltpu.make_async_copy(
            val_hbm.at[pos_buf.at[b]], rows, sem).start()

    @pl.when(nb > 0)
    def _():
        gather(0, rows0, sem_g0)

    def pair_body(i, _):
        b = i * 2
        pltpu.make_async_copy(
            val_hbm.at[pos_buf.at[b]], rows0, sem_g0).wait()
        s0 = pltpu.make_async_copy(
            rows0, out_hbm.at[row_buf.at[b]], sem_s0)
        s0.start()

        @pl.when(b + 1 < nb)
        def _():
            gather(b + 1, rows1, sem_g1)

        s0.wait()

        @pl.when(b + 1 < nb)
        def _():
            pltpu.make_async_copy(
                val_hbm.at[pos_buf.at[b + 1]], rows1, sem_g1).wait()
            s1 = pltpu.make_async_copy(
                rows1, out_hbm.at[row_buf.at[b + 1]], sem_s1)
            s1.start()

            @pl.when(b + 2 < nb)
            def _():
                gather(b + 2, rows0, sem_g0)

            s1.wait()

        return 0

    lax.fori_loop(0, pl.cdiv(nb, 2), pair_body, 0)


def kernel(mem, idx, val):
    M, D = mem.shape
    B, _ = val.shape
    assert B % (2 * L) == 0
    R = ((M + NW - 1) // NW + 7) // 8 * 8
    assert 0 < M - (NW - 1) * R <= R and M % 8 == 0
    NB_MAX = B // CHUNK

    mesh = plsc.VectorSubcoreMesh(
        core_axis_name="c", subcore_axis_name="s", num_cores=NC)

    sc = pl.kernel(
        functools.partial(_sc_body, M, B, R),
        out_type=jax.ShapeDtypeStruct((M, D), jnp.float32),
        mesh=mesh,
        compiler_params=pltpu.CompilerParams(needs_layout_passes=False),
        scratch_types=[
            pltpu.VMEM((B,), jnp.int32),
            pltpu.VMEM((NB_MAX, CHUNK), jnp.int32),
            pltpu.VMEM((NB_MAX, CHUNK), jnp.int32),
            pltpu.VMEM((R,), jnp.int32),
            pltpu.VMEM((L,), jnp.int32),
            pltpu.VMEM((L,), jnp.int32),
            pltpu.VMEM((CCH, D), jnp.float32),
            pltpu.VMEM((CCH, D), jnp.float32),
        ] + [pltpu.SemaphoreType.DMA] * 4,
    )
    return sc(mem, idx, val)
